# baseline (speedup 1.0000x reference)
import jax
import jax.numpy as jnp
from jax.experimental import pallas as pl
from jax.experimental.pallas import tpu as pltpu

E = 320000
N = 10000
E_TILE = 4000
N_TILE = 2000
GRID = E // E_TILE
N_STEPS = N // N_TILE


def _mlp_ln(parts, w1, slices, b1, w2, b2, gamma, beta):
    h = None
    for x, (lo, hi) in zip(parts, slices):
        p = jnp.dot(x, w1[lo:hi], preferred_element_type=jnp.float32)
        h = p if h is None else h + p
    h = jnp.maximum(h + b1, 0.0)
    h = jnp.dot(h, w2, preferred_element_type=jnp.float32) + b2
    h = jnp.maximum(h, 0.0)
    mu = jnp.mean(h, axis=-1, keepdims=True)
    var = jnp.mean(h * h, axis=-1, keepdims=True) - mu * mu
    s = jax.lax.rsqrt(var + 1e-5)
    return ((h - mu) * s) * gamma + beta


def _body(er_ref, es_ref, ee_ref, eg_ref,
          nn_ref, ng_ref, nr_ref, ns_ref,
          gn_ref, ge_ref, gg_ref,
          ew1_ref, eb1_ref, ew2_ref, eb2_ref, egam_ref, ebet_ref,
          nw1_ref, nb1_ref, nw2_ref, nb2_ref, ngam_ref, nbet_ref,
          gw1_ref, gb1_ref, gw2_ref, gb2_ref,
          eo_ref, no_ref, go_ref):
    i = pl.program_id(0)

    eo_ref[...] = _mlp_ln(
        (er_ref[...], es_ref[...], ee_ref[...], eg_ref[...]),
        ew1_ref[...], ((0, 128), (128, 256), (256, 272), (272, 400)),
        eb1_ref[...], ew2_ref[...], eb2_ref[...], egam_ref[...], ebet_ref[...])

    @pl.when(i < N_STEPS)
    def _node():
        no_ref[...] = _mlp_ln(
            (nn_ref[...], ng_ref[...], nr_ref[...], ns_ref[...]),
            nw1_ref[...], ((0, 128), (128, 256), (256, 384), (384, 512)),
            nb1_ref[...], nw2_ref[...], nb2_ref[...], ngam_ref[...],
            nbet_ref[...])

    @pl.when(i == 0)
    def _global():
        w1 = gw1_ref[...]
        h = jnp.dot(gn_ref[...], w1[0:128], preferred_element_type=jnp.float32)
        h = h + jnp.dot(ge_ref[...], w1[128:256],
                        preferred_element_type=jnp.float32)
        h = h + jnp.dot(gg_ref[...], w1[256:384],
                        preferred_element_type=jnp.float32)
        h = jnp.maximum(h + gb1_ref[...], 0.0)
        h = jnp.dot(h, gw2_ref[...],
                    preferred_element_type=jnp.float32) + gb2_ref[...]
        go_ref[...] = jnp.maximum(h, 0.0)


def _edge_spec(width):
    return pl.BlockSpec((E_TILE, width), lambda i: (i, 0))


def _node_spec(width):
    return pl.BlockSpec((N_TILE, width),
                        lambda i: (jnp.minimum(i, N_STEPS - 1), 0))


def _full_spec(shape):
    return pl.BlockSpec(shape, lambda i: tuple(0 for _ in shape))


def kernel(edge_attr, node_attr, global_attr, receiver_attr, sender_attr,
           global_attr_to_edge, global_attr_to_nodes, receiver_attr_to_nodes,
           sender_attr_to_node, node_attr_to_global, edge_attr_to_global,
           eW1, eb1, eW2, eb2, eg, ebt,
           nW1, nb1, nW2, nb2, ng, nbt,
           gW1, gb1, gW2, gb2):
    f32 = jnp.float32
    row = lambda v: v.reshape(1, -1)

    edge_out, node_out, global_out = pl.pallas_call(
        _body,
        grid=(GRID,),
        in_specs=[
            _edge_spec(128),
            _edge_spec(128),
            _edge_spec(16),
            _edge_spec(128),
            _node_spec(128),
            _node_spec(128),
            _node_spec(128),
            _node_spec(128),
            _full_spec((1, 128)),
            _full_spec((1, 128)),
            _full_spec((1, 128)),
            _full_spec((400, 128)),
            _full_spec((1, 128)),
            _full_spec((128, 128)),
            _full_spec((1, 128)),
            _full_spec((1, 128)),
            _full_spec((1, 128)),
            _full_spec((512, 128)),
            _full_spec((1, 128)),
            _full_spec((128, 128)),
            _full_spec((1, 128)),
            _full_spec((1, 128)),
            _full_spec((1, 128)),
            _full_spec((384, 128)),
            _full_spec((1, 128)),
            _full_spec((128, 128)),
            _full_spec((1, 128)),
        ],
        out_specs=[
            _edge_spec(128),
            _node_spec(128),
            _full_spec((1, 128)),
        ],
        out_shape=[
            jax.ShapeDtypeStruct((E, 128), f32),
            jax.ShapeDtypeStruct((N, 128), f32),
            jax.ShapeDtypeStruct((1, 128), f32),
        ],
        compiler_params=pltpu.CompilerParams(
            dimension_semantics=("arbitrary",)),
    )(receiver_attr, sender_attr, edge_attr, global_attr_to_edge,
      node_attr, global_attr_to_nodes, receiver_attr_to_nodes,
      sender_attr_to_node,
      row(node_attr_to_global), row(edge_attr_to_global), row(global_attr),
      eW1, row(eb1), eW2, row(eb2), row(eg), row(ebt),
      nW1, row(nb1), nW2, row(nb2), row(ng), row(nbt),
      gW1, row(gb1), gW2, row(gb2))

    return (edge_out, node_out, global_out)


# speedup vs baseline: 1.0364x; 1.0364x over previous
import jax
import jax.numpy as jnp
from jax.experimental import pallas as pl
from jax.experimental.pallas import tpu as pltpu

E = 320000
N = 10000
E_TILE = 8000
N_TILE = 1000
GRID = E // E_TILE
N_STEPS = N // N_TILE


def _mlp_ln(parts, w1, slices, b1, w2, b2, gamma, beta):
    h = None
    for x, (lo, hi) in zip(parts, slices):
        p = jnp.dot(x, w1[lo:hi], preferred_element_type=jnp.float32)
        h = p if h is None else h + p
    h = jnp.maximum(h + b1, 0.0)
    h = jnp.dot(h, w2, preferred_element_type=jnp.float32) + b2
    h = jnp.maximum(h, 0.0)
    mu = jnp.mean(h, axis=-1, keepdims=True)
    var = jnp.mean(h * h, axis=-1, keepdims=True) - mu * mu
    s = jax.lax.rsqrt(var + 1e-5)
    return ((h - mu) * s) * gamma + beta


def _body(er_ref, es_ref, ee_ref, eg_ref,
          nn_ref, ng_ref, nr_ref, ns_ref,
          gn_ref, ge_ref, gg_ref,
          ew1_ref, eb1_ref, ew2_ref, eb2_ref, egam_ref, ebet_ref,
          nw1_ref, nb1_ref, nw2_ref, nb2_ref, ngam_ref, nbet_ref,
          gw1_ref, gb1_ref, gw2_ref, gb2_ref,
          eo_ref, no_ref, go_ref):
    i = pl.program_id(0)

    eo_ref[...] = _mlp_ln(
        (er_ref[...], es_ref[...], ee_ref[...], eg_ref[...]),
        ew1_ref[...], ((0, 128), (128, 256), (256, 272), (272, 400)),
        eb1_ref[...], ew2_ref[...], eb2_ref[...], egam_ref[...], ebet_ref[...])

    @pl.when(i < N_STEPS)
    def _node():
        no_ref[...] = _mlp_ln(
            (nn_ref[...], ng_ref[...], nr_ref[...], ns_ref[...]),
            nw1_ref[...], ((0, 128), (128, 256), (256, 384), (384, 512)),
            nb1_ref[...], nw2_ref[...], nb2_ref[...], ngam_ref[...],
            nbet_ref[...])

    @pl.when(i == 0)
    def _global():
        w1 = gw1_ref[...]
        h = jnp.dot(gn_ref[...], w1[0:128], preferred_element_type=jnp.float32)
        h = h + jnp.dot(ge_ref[...], w1[128:256],
                        preferred_element_type=jnp.float32)
        h = h + jnp.dot(gg_ref[...], w1[256:384],
                        preferred_element_type=jnp.float32)
        h = jnp.maximum(h + gb1_ref[...], 0.0)
        h = jnp.dot(h, gw2_ref[...],
                    preferred_element_type=jnp.float32) + gb2_ref[...]
        go_ref[...] = jnp.maximum(h, 0.0)


def _edge_spec(width):
    return pl.BlockSpec((E_TILE, width), lambda i: (i, 0))


def _node_spec(width):
    return pl.BlockSpec((N_TILE, width),
                        lambda i: (jnp.minimum(i, N_STEPS - 1), 0))


def _full_spec(shape):
    return pl.BlockSpec(shape, lambda i: tuple(0 for _ in shape))


def kernel(edge_attr, node_attr, global_attr, receiver_attr, sender_attr,
           global_attr_to_edge, global_attr_to_nodes, receiver_attr_to_nodes,
           sender_attr_to_node, node_attr_to_global, edge_attr_to_global,
           eW1, eb1, eW2, eb2, eg, ebt,
           nW1, nb1, nW2, nb2, ng, nbt,
           gW1, gb1, gW2, gb2):
    f32 = jnp.float32
    row = lambda v: v.reshape(1, -1)

    edge_out, node_out, global_out = pl.pallas_call(
        _body,
        grid=(GRID,),
        in_specs=[
            _edge_spec(128),
            _edge_spec(128),
            _edge_spec(16),
            _edge_spec(128),
            _node_spec(128),
            _node_spec(128),
            _node_spec(128),
            _node_spec(128),
            _full_spec((1, 128)),
            _full_spec((1, 128)),
            _full_spec((1, 128)),
            _full_spec((400, 128)),
            _full_spec((1, 128)),
            _full_spec((128, 128)),
            _full_spec((1, 128)),
            _full_spec((1, 128)),
            _full_spec((1, 128)),
            _full_spec((512, 128)),
            _full_spec((1, 128)),
            _full_spec((128, 128)),
            _full_spec((1, 128)),
            _full_spec((1, 128)),
            _full_spec((1, 128)),
            _full_spec((384, 128)),
            _full_spec((1, 128)),
            _full_spec((128, 128)),
            _full_spec((1, 128)),
        ],
        out_specs=[
            _edge_spec(128),
            _node_spec(128),
            _full_spec((1, 128)),
        ],
        out_shape=[
            jax.ShapeDtypeStruct((E, 128), f32),
            jax.ShapeDtypeStruct((N, 128), f32),
            jax.ShapeDtypeStruct((1, 128), f32),
        ],
        compiler_params=pltpu.CompilerParams(
            dimension_semantics=("arbitrary",)),
    )(receiver_attr, sender_attr, edge_attr, global_attr_to_edge,
      node_attr, global_attr_to_nodes, receiver_attr_to_nodes,
      sender_attr_to_node,
      row(node_attr_to_global), row(edge_attr_to_global), row(global_attr),
      eW1, row(eb1), eW2, row(eb2), row(eg), row(ebt),
      nW1, row(nb1), nW2, row(nb2), row(ng), row(nbt),
      gW1, row(gb1), gW2, row(gb2))

    return (edge_out, node_out, global_out)
